# baseline (speedup 1.0000x reference)
import functools

import jax
import jax.numpy as jnp
from jax import lax
from jax.experimental import pallas as pl
from jax.experimental.pallas import tpu as pltpu
from jax.experimental.pallas import tpu_sc as plsc


NC = 2
NS = 16
NW = NC * NS
STAGES = (16384,)
CHUNK = 128


def _sc_gather_pairs(table, p_idx, q_idx, stage_base, n_pairs, emb_d):
    P = n_pairs // NW
    n_half = P // CHUNK
    n_ch = 2 * n_half
    mesh = plsc.VectorSubcoreMesh(core_axis_name="c", subcore_axis_name="s")

    @functools.partial(
        pl.kernel,
        mesh=mesh,
        out_type=jax.ShapeDtypeStruct((2 * n_pairs, emb_d), jnp.float32),
        scratch_types=[
            pltpu.VMEM((2 * P,), jnp.int32),
            pltpu.VMEM((CHUNK, emb_d), jnp.float32),
            pltpu.VMEM((CHUNK, emb_d), jnp.float32),
            pltpu.SemaphoreType.DMA,
            pltpu.SemaphoreType.DMA,
        ],
    )
    def gather_kernel(table_hbm, p_hbm, q_hbm, out_hbm, idx_v, rows_a, rows_b,
                      sem_a, sem_b):
        wid = lax.axis_index("s") * NC + lax.axis_index("c")
        src_off = stage_base + wid * P
        pltpu.sync_copy(p_hbm.at[pl.ds(src_off, P)], idx_v.at[pl.ds(0, P)])
        pltpu.sync_copy(q_hbm.at[pl.ds(src_off, P)], idx_v.at[pl.ds(P, P)])

        def out_off(c):
            is_q = (c >= n_half).astype(jnp.int32) if not isinstance(c, int) else int(c >= n_half)
            return wid * P + (c - n_half * is_q) * CHUNK + n_pairs * is_q

        pltpu.async_copy(
            table_hbm.at[idx_v.at[pl.ds(0, CHUNK)]], rows_a, sem_a
        ).wait()

        @pl.loop(0, n_ch - 1)
        def _(c):
            even = c % 2 == 0
            nxt_off = (c + 1) * CHUNK

            @pl.when(even)
            def _():
                pltpu.async_copy(
                    table_hbm.at[idx_v.at[pl.ds(nxt_off, CHUNK)]], rows_b, sem_b
                )
                pltpu.sync_copy(rows_a, out_hbm.at[pl.ds(out_off(c), CHUNK)])
                pltpu.make_async_copy(
                    table_hbm.at[idx_v.at[pl.ds(nxt_off, CHUNK)]], rows_b, sem_b
                ).wait()

            @pl.when(jnp.logical_not(even))
            def _():
                pltpu.async_copy(
                    table_hbm.at[idx_v.at[pl.ds(nxt_off, CHUNK)]], rows_a, sem_a
                )
                pltpu.sync_copy(rows_b, out_hbm.at[pl.ds(out_off(c), CHUNK)])
                pltpu.make_async_copy(
                    table_hbm.at[idx_v.at[pl.ds(nxt_off, CHUNK)]], rows_a, sem_a
                ).wait()

        last = n_ch - 1
        last_buf = rows_a if last % 2 == 0 else rows_b
        pltpu.sync_copy(last_buf, out_hbm.at[pl.ds(out_off(last), CHUNK)])

    return gather_kernel(table, p_idx, q_idx)


def _mlp_body(ei_ref, ej_ref, w1_ref, b1_ref, w2_ref, b2_ref, o_ref):
    d = ei_ref[...] - ej_ref[...]
    d = (d * d).astype(jnp.bfloat16)
    h = jnp.dot(d, w1_ref[...], preferred_element_type=jnp.float32)
    h = jnp.maximum(h + b1_ref[...], 0.0).astype(jnp.bfloat16)
    pred = jnp.dot(h, w2_ref[...], preferred_element_type=jnp.float32) + b2_ref[...]
    o_ref[...] = pred.reshape(1, 1, pred.shape[0])


def _mlp(gathered, W1b, b1_2d, W2b, b2_2d, n_pairs, emb_d, hid):
    BLK = 4096 if n_pairs % 4096 == 0 else 2048
    nb = n_pairs // BLK
    return pl.pallas_call(
        _mlp_body,
        grid=(nb,),
        in_specs=[
            pl.BlockSpec((BLK, emb_d), lambda i: (i, 0)),
            pl.BlockSpec((BLK, emb_d), lambda i, _nb=nb: (i + _nb, 0)),
            pl.BlockSpec((emb_d, hid), lambda i: (0, 0)),
            pl.BlockSpec((1, hid), lambda i: (0, 0)),
            pl.BlockSpec((hid, 1), lambda i: (0, 0)),
            pl.BlockSpec((1, 1), lambda i: (0, 0)),
        ],
        out_specs=pl.BlockSpec((1, 1, BLK), lambda i: (i, 0, 0)),
        out_shape=jax.ShapeDtypeStruct((nb, 1, BLK), jnp.float32),
    )(gathered, gathered, W1b, b1_2d, W2b, b2_2d)


def kernel(embds, W1, b1, W2, b2, p_vertices, q_vertices):
    B = p_vertices.shape[0]
    emb_d = embds.shape[1]
    hid = W1.shape[1]
    W1b = W1.astype(jnp.bfloat16)
    W2b = W2.astype(jnp.bfloat16)
    b1_2d = b1.reshape(1, hid)
    b2_2d = b2.reshape(1, 1)

    outs = []
    base = 0
    for n_pairs in STAGES:
        gathered = _sc_gather_pairs(embds, p_vertices, q_vertices, base, n_pairs, emb_d)
        outs.append(_mlp(gathered, W1b, b1_2d, W2b, b2_2d, n_pairs, emb_d, hid).reshape(n_pairs))
        base += n_pairs
    if len(outs) == 1:
        return outs[0]
    return jnp.concatenate(outs)


# speedup vs baseline: 1.0299x; 1.0299x over previous
import functools

import jax
import jax.numpy as jnp
from jax import lax
from jax.experimental import pallas as pl
from jax.experimental.pallas import tpu as pltpu
from jax.experimental.pallas import tpu_sc as plsc


NC = 2
NS = 16
NW = NC * NS
STAGES = (16384,)
CHUNK = 128


def _sc_gather_pairs(table, p_idx, q_idx, stage_base, n_pairs, emb_d):
    P = n_pairs // NW
    n_half = P // CHUNK
    n_ch = 2 * n_half
    mesh = plsc.VectorSubcoreMesh(core_axis_name="c", subcore_axis_name="s")

    @functools.partial(
        pl.kernel,
        mesh=mesh,
        out_type=jax.ShapeDtypeStruct((2 * n_pairs, emb_d), jnp.float32),
        scratch_types=[
            pltpu.VMEM((2 * P,), jnp.int32),
            pltpu.VMEM((CHUNK, emb_d), jnp.float32),
            pltpu.VMEM((CHUNK, emb_d), jnp.float32),
            pltpu.VMEM((CHUNK, emb_d), jnp.float32),
            pltpu.SemaphoreType.DMA,
            pltpu.SemaphoreType.DMA,
            pltpu.SemaphoreType.DMA,
            pltpu.SemaphoreType.DMA,
            pltpu.SemaphoreType.DMA,
            pltpu.SemaphoreType.DMA,
        ],
    )
    def gather_kernel(table_hbm, p_hbm, q_hbm, out_hbm, idx_v,
                      rows_a, rows_b, rows_c, ga, gb, gc, wa, wb, wc):
        wid = lax.axis_index("s") * NC + lax.axis_index("c")
        src_off = stage_base + wid * P
        pltpu.sync_copy(p_hbm.at[pl.ds(src_off, P)], idx_v.at[pl.ds(0, P)])
        pltpu.sync_copy(q_hbm.at[pl.ds(src_off, P)], idx_v.at[pl.ds(P, P)])

        bufs = (rows_a, rows_b, rows_c)
        gsems = (ga, gb, gc)
        wsems = (wa, wb, wc)

        def out_off(c):
            is_q = int(c >= n_half)
            return wid * P + (c - n_half * is_q) * CHUNK + n_pairs * is_q

        def g_src(c):
            return table_hbm.at[idx_v.at[pl.ds(c * CHUNK, CHUNK)]]

        def w_dst(c):
            return out_hbm.at[pl.ds(out_off(c), CHUNK)]

        for c in range(min(3, n_ch)):
            pltpu.async_copy(g_src(c), bufs[c], gsems[c])
        for c in range(n_ch):
            i = c % 3
            pltpu.make_async_copy(g_src(c), bufs[i], gsems[i]).wait()
            pltpu.async_copy(bufs[i], w_dst(c), wsems[i])
            if c + 3 < n_ch:
                pltpu.make_async_copy(bufs[i], w_dst(c), wsems[i]).wait()
                pltpu.async_copy(g_src(c + 3), bufs[i], gsems[i])
        for c in range(max(0, n_ch - 3), n_ch):
            i = c % 3
            pltpu.make_async_copy(bufs[i], w_dst(c), wsems[i]).wait()

    return gather_kernel(table, p_idx, q_idx)


def _mlp_body(ei_ref, ej_ref, w1_ref, b1_ref, w2_ref, b2_ref, o_ref):
    d = ei_ref[...] - ej_ref[...]
    d = (d * d).astype(jnp.bfloat16)
    h = jnp.dot(d, w1_ref[...], preferred_element_type=jnp.float32)
    h = jnp.maximum(h + b1_ref[...], 0.0).astype(jnp.bfloat16)
    pred = jnp.dot(h, w2_ref[...], preferred_element_type=jnp.float32) + b2_ref[...]
    o_ref[...] = pred.reshape(1, 1, pred.shape[0])


def _mlp(gathered, W1b, b1_2d, W2b, b2_2d, n_pairs, emb_d, hid):
    BLK = 4096 if n_pairs % 4096 == 0 else 2048
    nb = n_pairs // BLK
    return pl.pallas_call(
        _mlp_body,
        grid=(nb,),
        in_specs=[
            pl.BlockSpec((BLK, emb_d), lambda i: (i, 0)),
            pl.BlockSpec((BLK, emb_d), lambda i, _nb=nb: (i + _nb, 0)),
            pl.BlockSpec((emb_d, hid), lambda i: (0, 0)),
            pl.BlockSpec((1, hid), lambda i: (0, 0)),
            pl.BlockSpec((hid, 1), lambda i: (0, 0)),
            pl.BlockSpec((1, 1), lambda i: (0, 0)),
        ],
        out_specs=pl.BlockSpec((1, 1, BLK), lambda i: (i, 0, 0)),
        out_shape=jax.ShapeDtypeStruct((nb, 1, BLK), jnp.float32),
    )(gathered, gathered, W1b, b1_2d, W2b, b2_2d)


def kernel(embds, W1, b1, W2, b2, p_vertices, q_vertices):
    B = p_vertices.shape[0]
    emb_d = embds.shape[1]
    hid = W1.shape[1]
    W1b = W1.astype(jnp.bfloat16)
    W2b = W2.astype(jnp.bfloat16)
    b1_2d = b1.reshape(1, hid)
    b2_2d = b2.reshape(1, 1)

    outs = []
    base = 0
    for n_pairs in STAGES:
        gathered = _sc_gather_pairs(embds, p_vertices, q_vertices, base, n_pairs, emb_d)
        outs.append(_mlp(gathered, W1b, b1_2d, W2b, b2_2d, n_pairs, emb_d, hid).reshape(n_pairs))
        base += n_pairs
    if len(outs) == 1:
        return outs[0]
    return jnp.concatenate(outs)
